# Initial kernel scaffold; baseline (speedup 1.0000x reference)
import jax
import jax.numpy as jnp
from jax.experimental import pallas as pl


def kernel(x, edge_index, edge_attr, batch, W_enc, b_enc, W_msg, W_edge, b_msg, W_upd, W_self, b_upd, W_dec, b_dec):
    raise NotImplementedError("write your pallas kernel here")


# speedup vs baseline: 2.4313x; 2.4313x over previous
import functools

import jax
import jax.numpy as jnp
from jax import lax
from jax.experimental import pallas as pl
from jax.experimental.pallas import tpu as pltpu
from jax.experimental.pallas import tpu_sc as plsc

N, E, D, H, DE, T = 10000, 320000, 128, 128, 16, 4

NPAD = 10240
CHUNK = 128
NCORES, NSUB = 2, 16
NTILES = NCORES * NSUB
CHUNKS_PER_TILE = -(-E // (NTILES * CHUNK))
EPT = CHUNK * CHUNKS_PER_TILE
EPAD = NTILES * EPT
ROWS_PER_TILE = NPAD // NSUB
RB = 1000
EB = 2048


def _dot(a, b):
    return jnp.dot(a, b, preferred_element_type=jnp.float32)


def _tc_pre_body(x_ref, we_ref, be_ref, wmx_ref, wmh_ref, wsx_ref, bu_ref,
                 xin_ref, hpx_ref, hp_ref, sxb_ref):
    xin = jnp.maximum(_dot(x_ref[...], we_ref[...]) + be_ref[...], 0.0)
    xin_ref[...] = xin
    hpx = _dot(xin, wmx_ref[...])
    hpx_ref[...] = hpx
    hp_ref[...] = hpx + _dot(xin, wmh_ref[...])
    sxb_ref[...] = _dot(xin, wsx_ref[...]) + bu_ref[...]


def _tc_pre(x, We, be, Wmx, Wmh, Wsx, bu):
    wspec = pl.BlockSpec((D, H), lambda i: (0, 0))
    bspec = pl.BlockSpec((1, H), lambda i: (0, 0))
    rspec = pl.BlockSpec((RB, D), lambda i: (i, 0))
    return pl.pallas_call(
        _tc_pre_body,
        grid=(N // RB,),
        in_specs=[rspec, wspec, bspec, wspec, wspec, wspec, bspec],
        out_specs=[pl.BlockSpec((RB, H), lambda i: (i, 0))] * 4,
        out_shape=[jax.ShapeDtypeStruct((N, H), jnp.float32)] * 4,
    )(x, We, be, Wmx, Wmh, Wsx, bu)


def _tc_epb_body(ea_ref, we_ref, bm_ref, epb_ref):
    epb_ref[...] = _dot(ea_ref[...], we_ref[...]) + bm_ref[...]


def _tc_epb(ea_p, W_edge, bm):
    return pl.pallas_call(
        _tc_epb_body,
        grid=(EPAD // EB,),
        in_specs=[pl.BlockSpec((EB, DE), lambda i: (i, 0)),
                  pl.BlockSpec((DE, H), lambda i: (0, 0)),
                  pl.BlockSpec((1, H), lambda i: (0, 0))],
        out_specs=pl.BlockSpec((EB, H), lambda i: (i, 0)),
        out_shape=jax.ShapeDtypeStruct((EPAD, H), jnp.float32),
    )(ea_p, W_edge, bm)


def _tc_step_body(aggp_ref, h_ref, hpx_ref, sxb_ref, wu_ref, wsh_ref, wmh_ref,
                  h2_ref, hp2_ref):
    agg = aggp_ref[0] + aggp_ref[1]
    h2 = jnp.maximum(
        _dot(agg, wu_ref[...]) + _dot(h_ref[...], wsh_ref[...]) + sxb_ref[...],
        0.0)
    h2_ref[...] = h2
    hp2_ref[...] = hpx_ref[...] + _dot(h2, wmh_ref[...])


def _tc_step(aggp, h, hpx, sxb, W_upd, Wsh, Wmh):
    wspec = pl.BlockSpec((H, H), lambda i: (0, 0))
    rspec = pl.BlockSpec((RB, H), lambda i: (i, 0))
    return pl.pallas_call(
        _tc_step_body,
        grid=(N // RB,),
        in_specs=[pl.BlockSpec((NCORES, RB, H), lambda i: (0, i, 0)),
                  rspec, rspec, rspec, wspec, wspec, wspec],
        out_specs=[rspec, rspec],
        out_shape=[jax.ShapeDtypeStruct((N, H), jnp.float32)] * 2,
    )(aggp, h, hpx, sxb, W_upd, Wsh, Wmh)


def _tc_out_body(xin_ref, h_ref, wdx_ref, wdh_ref, bd_ref, out_ref):
    out_ref[...] = (_dot(xin_ref[...], wdx_ref[...]) +
                    _dot(h_ref[...], wdh_ref[...]) + bd_ref[...])


def _tc_out(xin, h, Wdx, Wdh, bd):
    wspec = pl.BlockSpec((H, D), lambda i: (0, 0))
    rspec = pl.BlockSpec((RB, H), lambda i: (i, 0))
    return pl.pallas_call(
        _tc_out_body,
        grid=(N // RB,),
        in_specs=[rspec, rspec, wspec, wspec, pl.BlockSpec((1, D), lambda i: (0, 0))],
        out_specs=pl.BlockSpec((RB, D), lambda i: (i, 0)),
        out_shape=jax.ShapeDtypeStruct((N, D), jnp.float32),
    )(xin, h, Wdx, Wdh, bd)


def _sc_agg_body(hp_hbm, src_hbm, dst_hbm, epb_hbm, out_hbm,
                 zbuf, idxs, idxd, rows, epbv, agg_sh, sem):
    c = lax.axis_index("c")
    s = lax.axis_index("s")

    def _z(j, carry):
        for l in range(H // 16):
            zbuf[j, pl.ds(l * 16, 16)] = jnp.zeros((16,), jnp.float32)
        return carry
    lax.fori_loop(0, 32, _z, 0)

    def _zs(k, carry):
        pltpu.sync_copy(zbuf, agg_sh.at[pl.ds(s * ROWS_PER_TILE + k * 32, 32)])
        return carry
    lax.fori_loop(0, ROWS_PER_TILE // 32, _zs, 0)
    plsc.subcore_barrier()

    base = c * (EPAD // NCORES) + s * EPT

    def _chunk(g, carry):
        eb = base + g * CHUNK
        pltpu.sync_copy(src_hbm.at[pl.ds(eb, CHUNK)], idxs)
        pltpu.sync_copy(dst_hbm.at[pl.ds(eb, CHUNK)], idxd)
        pltpu.async_copy(hp_hbm.at[idxs], rows, sem).wait()
        pltpu.sync_copy(epb_hbm.at[pl.ds(eb, CHUNK)], epbv)

        def _row(i, c2):
            for l in range(H // 16):
                sl = pl.ds(l * 16, 16)
                rows[i, sl] = jnp.maximum(rows[i, sl] + epbv[i, sl], 0.0)
            return c2
        lax.fori_loop(0, CHUNK, _row, 0)
        pltpu.sync_copy(rows, agg_sh.at[idxd], add=True)
        return carry
    lax.fori_loop(0, CHUNKS_PER_TILE, _chunk, 0)

    plsc.subcore_barrier()
    pltpu.sync_copy(agg_sh.at[pl.ds(s * ROWS_PER_TILE, ROWS_PER_TILE)],
                    out_hbm.at[c, pl.ds(s * ROWS_PER_TILE, ROWS_PER_TILE)])


@functools.cache
def _make_sc_agg():
    return functools.partial(
        pl.kernel,
        out_type=jax.ShapeDtypeStruct((NCORES, NPAD, H), jnp.float32),
        mesh=plsc.VectorSubcoreMesh(core_axis_name="c", subcore_axis_name="s"),
        scratch_types=[
            pltpu.VMEM((32, H), jnp.float32),
            pltpu.VMEM((CHUNK,), jnp.int32),
            pltpu.VMEM((CHUNK,), jnp.int32),
            pltpu.VMEM((CHUNK, H), jnp.float32),
            pltpu.VMEM((CHUNK, H), jnp.float32),
            pltpu.VMEM_SHARED((NPAD, H), jnp.float32),
            pltpu.SemaphoreType.DMA,
        ],
    )(_sc_agg_body)


def kernel(x, edge_index, edge_attr, batch, W_enc, b_enc, W_msg, W_edge, b_msg,
           W_upd, W_self, b_upd, W_dec, b_dec):
    f32 = jnp.float32
    pad = EPAD - E
    src_p = jnp.concatenate([edge_index[0], jnp.zeros((pad,), jnp.int32)])
    dst_p = jnp.concatenate([edge_index[1], jnp.full((pad,), N, jnp.int32)])
    ea_p = jnp.concatenate([edge_attr, jnp.zeros((pad, DE), f32)])

    Wmx, Wmh = W_msg[:H], W_msg[H:2 * H] + W_msg[2 * H:]
    Wsx, Wsh = W_self[:H], W_self[H:2 * H] + W_self[2 * H:]
    Wdx, Wdh = W_dec[:H], W_dec[H:]
    be, bm = b_enc.reshape(1, H), b_msg.reshape(1, H)
    bu, bd = b_upd.reshape(1, H), b_dec.reshape(1, D)

    xin, hpx, hp, sxb = _tc_pre(x, W_enc, be, Wmx, Wmh, Wsx, bu)
    epb = _tc_epb(ea_p, W_edge, bm)

    sc_agg = _make_sc_agg()
    h = xin
    for _ in range(T):
        aggp = sc_agg(hp, src_p, dst_p, epb)
        h, hp = _tc_step(aggp, h, hpx, sxb, W_upd, Wsh, Wmh)

    out = _tc_out(xin, h, Wdx, Wdh, bd)
    return (out, h)


# speedup vs baseline: 3.0159x; 1.2404x over previous
import functools

import jax
import jax.numpy as jnp
from jax import lax
from jax.experimental import pallas as pl
from jax.experimental.pallas import tpu as pltpu
from jax.experimental.pallas import tpu_sc as plsc

N, E, D, H, DE, T = 10000, 320000, 128, 128, 16, 4

NPAD = 10240
CHUNK = 64
NCORES, NSUB = 2, 16
NTILES = NCORES * NSUB
CHUNKS_PER_TILE = 158
PAIRS = CHUNKS_PER_TILE // 2
EPT = CHUNK * CHUNKS_PER_TILE
EPAD = NTILES * EPT
ROWS_PER_TILE = NPAD // NSUB
RB = 1000
EB = 2048


def _dot(a, b):
    return jnp.dot(a, b, preferred_element_type=jnp.float32)


def _tc_pre_body(x_ref, we_ref, be_ref, wmx_ref, wmh_ref, wsx_ref, bu_ref,
                 xin_ref, hpx_ref, hp_ref, sxb_ref):
    xin = jnp.maximum(_dot(x_ref[...], we_ref[...]) + be_ref[...], 0.0)
    xin_ref[...] = xin
    hpx = _dot(xin, wmx_ref[...])
    hpx_ref[...] = hpx
    hp_ref[...] = hpx + _dot(xin, wmh_ref[...])
    sxb_ref[...] = _dot(xin, wsx_ref[...]) + bu_ref[...]


def _tc_pre(x, We, be, Wmx, Wmh, Wsx, bu):
    wspec = pl.BlockSpec((D, H), lambda i: (0, 0))
    bspec = pl.BlockSpec((1, H), lambda i: (0, 0))
    rspec = pl.BlockSpec((RB, D), lambda i: (i, 0))
    return pl.pallas_call(
        _tc_pre_body,
        grid=(N // RB,),
        in_specs=[rspec, wspec, bspec, wspec, wspec, wspec, bspec],
        out_specs=[pl.BlockSpec((RB, H), lambda i: (i, 0))] * 4,
        out_shape=[jax.ShapeDtypeStruct((N, H), jnp.float32)] * 4,
    )(x, We, be, Wmx, Wmh, Wsx, bu)


def _tc_epb_body(ea_ref, we_ref, bm_ref, epb_ref):
    epb_ref[...] = _dot(ea_ref[...], we_ref[...]) + bm_ref[...]


def _tc_epb(ea_p, W_edge, bm):
    return pl.pallas_call(
        _tc_epb_body,
        grid=(EPAD // EB,),
        in_specs=[pl.BlockSpec((EB, DE), lambda i: (i, 0)),
                  pl.BlockSpec((DE, H), lambda i: (0, 0)),
                  pl.BlockSpec((1, H), lambda i: (0, 0))],
        out_specs=pl.BlockSpec((EB, H), lambda i: (i, 0)),
        out_shape=jax.ShapeDtypeStruct((EPAD, H), jnp.float32),
    )(ea_p, W_edge, bm)


def _tc_step_body(aggp_ref, h_ref, hpx_ref, sxb_ref, wu_ref, wsh_ref, wmh_ref,
                  h2_ref, hp2_ref):
    agg = aggp_ref[0] + aggp_ref[1]
    h2 = jnp.maximum(
        _dot(agg, wu_ref[...]) + _dot(h_ref[...], wsh_ref[...]) + sxb_ref[...],
        0.0)
    h2_ref[...] = h2
    hp2_ref[...] = hpx_ref[...] + _dot(h2, wmh_ref[...])


def _tc_step(aggp, h, hpx, sxb, W_upd, Wsh, Wmh):
    wspec = pl.BlockSpec((H, H), lambda i: (0, 0))
    rspec = pl.BlockSpec((RB, H), lambda i: (i, 0))
    return pl.pallas_call(
        _tc_step_body,
        grid=(N // RB,),
        in_specs=[pl.BlockSpec((NCORES, RB, H), lambda i: (0, i, 0)),
                  rspec, rspec, rspec, wspec, wspec, wspec],
        out_specs=[rspec, rspec],
        out_shape=[jax.ShapeDtypeStruct((N, H), jnp.float32)] * 2,
    )(aggp, h, hpx, sxb, W_upd, Wsh, Wmh)


def _tc_out_body(xin_ref, h_ref, wdx_ref, wdh_ref, bd_ref, out_ref):
    out_ref[...] = (_dot(xin_ref[...], wdx_ref[...]) +
                    _dot(h_ref[...], wdh_ref[...]) + bd_ref[...])


def _tc_out(xin, h, Wdx, Wdh, bd):
    wspec = pl.BlockSpec((H, D), lambda i: (0, 0))
    rspec = pl.BlockSpec((RB, H), lambda i: (i, 0))
    return pl.pallas_call(
        _tc_out_body,
        grid=(N // RB,),
        in_specs=[rspec, rspec, wspec, wspec, pl.BlockSpec((1, D), lambda i: (0, 0))],
        out_specs=pl.BlockSpec((RB, D), lambda i: (i, 0)),
        out_shape=jax.ShapeDtypeStruct((N, D), jnp.float32),
    )(xin, h, Wdx, Wdh, bd)


def _sc_agg_body(hp_hbm, src_hbm, dst_hbm, epb_hbm, out_hbm,
                 zbuf, idxs0, idxd0, idxs1, idxd1,
                 rows0, epbv0, rows1, epbv1, agg_sh,
                 semg0, seme0, semsc0, semg1, seme1, semsc1):
    c = lax.axis_index("c")
    s = lax.axis_index("s")

    def _z(j, carry):
        for l in range(H // 16):
            zbuf[j, pl.ds(l * 16, 16)] = jnp.zeros((16,), jnp.float32)
        return carry
    lax.fori_loop(0, 32, _z, 0)

    def _zs(k, carry):
        pltpu.sync_copy(zbuf, agg_sh.at[pl.ds(s * ROWS_PER_TILE + k * 32, 32)])
        return carry
    lax.fori_loop(0, ROWS_PER_TILE // 32, _zs, 0)
    plsc.subcore_barrier()

    base = c * (EPAD // NCORES) + s * EPT

    def load_idx(eb, is_, id_):
        pltpu.sync_copy(src_hbm.at[pl.ds(eb, CHUNK)], is_)
        pltpu.sync_copy(dst_hbm.at[pl.ds(eb, CHUNK)], id_)

    def start_fetch(eb, is_, rows, epbv, semg, seme):
        pltpu.async_copy(hp_hbm.at[is_], rows, semg)
        pltpu.async_copy(epb_hbm.at[pl.ds(eb, CHUNK)], epbv, seme)

    def wait_fetch(is_, rows, epbv, semg, seme):
        pltpu.make_async_copy(hp_hbm.at[is_], rows, semg).wait()
        pltpu.make_async_copy(epb_hbm.at[pl.ds(0, CHUNK)], epbv, seme).wait()

    def compute(rows, epbv):
        def _row(i, c2):
            for l in range(H // 16):
                sl = pl.ds(l * 16, 16)
                rows[i, sl] = jnp.maximum(rows[i, sl] + epbv[i, sl], 0.0)
            return c2
        lax.fori_loop(0, CHUNK, _row, 0)

    def start_scatter(rows, id_, semsc):
        pltpu.async_copy(rows, agg_sh.at[id_], semsc, add=True)

    def wait_scatter(rows, id_, semsc):
        pltpu.make_async_copy(rows, agg_sh.at[id_], semsc).wait()

    load_idx(base, idxs0, idxd0)
    start_fetch(base, idxs0, rows0, epbv0, semg0, seme0)

    def pair(p, carry):
        g1 = base + (2 * p + 1) * CHUNK
        g2 = base + (2 * p + 2) * CHUNK

        @pl.when(p > 0)
        def _():
            wait_scatter(rows1, idxd1, semsc1)

        load_idx(g1, idxs1, idxd1)
        start_fetch(g1, idxs1, rows1, epbv1, semg1, seme1)

        wait_fetch(idxs0, rows0, epbv0, semg0, seme0)
        compute(rows0, epbv0)
        start_scatter(rows0, idxd0, semsc0)

        wait_fetch(idxs1, rows1, epbv1, semg1, seme1)
        compute(rows1, epbv1)
        start_scatter(rows1, idxd1, semsc1)

        wait_scatter(rows0, idxd0, semsc0)

        @pl.when(p < PAIRS - 1)
        def _():
            load_idx(g2, idxs0, idxd0)
            start_fetch(g2, idxs0, rows0, epbv0, semg0, seme0)
        return carry

    lax.fori_loop(0, PAIRS, pair, 0)
    wait_scatter(rows1, idxd1, semsc1)

    plsc.subcore_barrier()
    pltpu.sync_copy(agg_sh.at[pl.ds(s * ROWS_PER_TILE, ROWS_PER_TILE)],
                    out_hbm.at[c, pl.ds(s * ROWS_PER_TILE, ROWS_PER_TILE)])


@functools.cache
def _make_sc_agg():
    return functools.partial(
        pl.kernel,
        out_type=jax.ShapeDtypeStruct((NCORES, NPAD, H), jnp.float32),
        mesh=plsc.VectorSubcoreMesh(core_axis_name="c", subcore_axis_name="s"),
        scratch_types=[
            pltpu.VMEM((32, H), jnp.float32),
            pltpu.VMEM((CHUNK,), jnp.int32),
            pltpu.VMEM((CHUNK,), jnp.int32),
            pltpu.VMEM((CHUNK,), jnp.int32),
            pltpu.VMEM((CHUNK,), jnp.int32),
            pltpu.VMEM((CHUNK, H), jnp.float32),
            pltpu.VMEM((CHUNK, H), jnp.float32),
            pltpu.VMEM((CHUNK, H), jnp.float32),
            pltpu.VMEM((CHUNK, H), jnp.float32),
            pltpu.VMEM_SHARED((NPAD, H), jnp.float32),
            pltpu.SemaphoreType.DMA,
            pltpu.SemaphoreType.DMA,
            pltpu.SemaphoreType.DMA,
            pltpu.SemaphoreType.DMA,
            pltpu.SemaphoreType.DMA,
            pltpu.SemaphoreType.DMA,
        ],
    )(_sc_agg_body)


def kernel(x, edge_index, edge_attr, batch, W_enc, b_enc, W_msg, W_edge, b_msg,
           W_upd, W_self, b_upd, W_dec, b_dec):
    f32 = jnp.float32
    pad = EPAD - E
    src_p = jnp.concatenate([edge_index[0], jnp.zeros((pad,), jnp.int32)])
    dst_p = jnp.concatenate([edge_index[1], jnp.full((pad,), N, jnp.int32)])
    ea_p = jnp.concatenate([edge_attr, jnp.zeros((pad, DE), f32)])

    Wmx, Wmh = W_msg[:H], W_msg[H:2 * H] + W_msg[2 * H:]
    Wsx, Wsh = W_self[:H], W_self[H:2 * H] + W_self[2 * H:]
    Wdx, Wdh = W_dec[:H], W_dec[H:]
    be, bm = b_enc.reshape(1, H), b_msg.reshape(1, H)
    bu, bd = b_upd.reshape(1, H), b_dec.reshape(1, D)

    xin, hpx, hp, sxb = _tc_pre(x, W_enc, be, Wmx, Wmh, Wsx, bu)
    epb = _tc_epb(ea_p, W_edge, bm)

    sc_agg = _make_sc_agg()
    h = xin
    for _ in range(T):
        aggp = sc_agg(hp, src_p, dst_p, epb)
        h, hp = _tc_step(aggp, h, hpx, sxb, W_upd, Wsh, Wmh)

    out = _tc_out(xin, h, Wdx, Wdh, bd)
    return (out, h)
